# baseline (speedup 1.0000x reference)
import functools

import jax
import jax.numpy as jnp
from jax import lax
from jax.experimental import pallas as pl
from jax.experimental.pallas import tpu as pltpu
from jax.experimental.pallas import tpu_sc as plsc

D = 32
L = 16
NW = 32
CHUNK = 128


def _rsqrt(n):
    i = plsc.bitcast(n, jnp.int32)
    i = jnp.int32(0x5F3759DF) - (i >> 1)
    y = plsc.bitcast(i, jnp.float32)
    for _ in range(3):
        y = y * (jnp.float32(1.5) - jnp.float32(0.5) * n * y * y)
    return y


def kernel(x, weight):
    B0, S = x.shape
    N = B0 * S
    per_w = N // NW
    n_chunks = per_w // CHUNK
    xf = x.reshape(NW * n_chunks, CHUNK)

    mesh = plsc.VectorSubcoreMesh(core_axis_name="c", subcore_axis_name="s")

    @functools.partial(
        pl.kernel,
        mesh=mesh,
        compiler_params=pltpu.CompilerParams(
            needs_layout_passes=False, use_tc_tiling_on_sc=False
        ),
        out_type=jax.ShapeDtypeStruct((N, D), jnp.float32),
        scratch_types=[
            pltpu.VMEM((n_chunks, CHUNK), jnp.int32),
            pltpu.VMEM((CHUNK, D), jnp.float32),
            pltpu.SemaphoreType.DMA,
        ],
    )
    def run(x_hbm, tab_hbm, out_hbm, idx_v, rows_v, sem):
        wid = lax.axis_index("s") * 2 + lax.axis_index("c")
        pltpu.sync_copy(x_hbm.at[pl.ds(wid * n_chunks, n_chunks)], idx_v)
        lanes = lax.iota(jnp.int32, 16)

        def chunk_body(j, carry):
            pltpu.async_copy(tab_hbm.at[idx_v.at[j]], rows_v, sem).wait()
            for g in range(CHUNK // L):
                rows = lanes + jnp.int32(g * L)
                vs = []
                acc = jnp.zeros((L,), jnp.float32)
                for d in range(D):
                    col = jnp.full((L,), d, jnp.int32)
                    v = plsc.load_gather(rows_v, [rows, col])
                    acc = acc + v * v
                    vs.append(v)
                y = _rsqrt(jnp.maximum(acc, jnp.float32(1e-24)))
                for d in range(D):
                    col = jnp.full((L,), d, jnp.int32)
                    plsc.store_scatter(rows_v, [rows, col], vs[d] * y)
            pltpu.sync_copy(
                rows_v, out_hbm.at[pl.ds((wid * n_chunks + j) * CHUNK, CHUNK)]
            )
            return carry

        lax.fori_loop(0, n_chunks, chunk_body, jnp.int32(0))

    out = run(xf, weight)
    return out.reshape(B0, S, D)


# speedup vs baseline: 1.0486x; 1.0486x over previous
import functools

import jax
import jax.numpy as jnp
from jax import lax
from jax.experimental import pallas as pl
from jax.experimental.pallas import tpu as pltpu
from jax.experimental.pallas import tpu_sc as plsc

D = 32
L = 16
NW = 32
CHUNK = 128


def _rsqrt(n):
    i = plsc.bitcast(n, jnp.int32)
    i = jnp.int32(0x5F3759DF) - (i >> 1)
    y = plsc.bitcast(i, jnp.float32)
    for _ in range(3):
        y = y * (jnp.float32(1.5) - jnp.float32(0.5) * n * y * y)
    return y


def kernel(x, weight):
    B0, S = x.shape
    N = B0 * S
    per_w = N // NW
    n_chunks = per_w // CHUNK
    xf = x.reshape(NW * n_chunks, CHUNK)

    mesh = plsc.VectorSubcoreMesh(core_axis_name="c", subcore_axis_name="s")

    NB = 8
    n_rounds = n_chunks // NB

    @functools.partial(
        pl.kernel,
        mesh=mesh,
        compiler_params=pltpu.CompilerParams(
            needs_layout_passes=False, use_tc_tiling_on_sc=False
        ),
        out_type=jax.ShapeDtypeStruct((N, D), jnp.float32),
        scratch_types=(
            [pltpu.VMEM((n_chunks, CHUNK), jnp.int32)]
            + [pltpu.VMEM((CHUNK, D), jnp.float32) for _ in range(NB)]
            + [pltpu.SemaphoreType.DMA for _ in range(2 * NB)]
        ),
    )
    def run(x_hbm, tab_hbm, out_hbm, idx_v, *scratch):
        bufs = scratch[:NB]
        sin = scratch[NB : 2 * NB]
        sout = scratch[2 * NB :]
        wid = lax.axis_index("s") * 2 + lax.axis_index("c")
        pltpu.sync_copy(x_hbm.at[pl.ds(wid * n_chunks, n_chunks)], idx_v)
        lanes = lax.iota(jnp.int32, 16)

        def normalize(buf):
            for g in range(CHUNK // L):
                rows = lanes + jnp.int32(g * L)
                vs = []
                acc = jnp.zeros((L,), jnp.float32)
                for d in range(D):
                    col = jnp.full((L,), d, jnp.int32)
                    v = plsc.load_gather(buf, [rows, col])
                    acc = acc + v * v
                    vs.append(v)
                y = _rsqrt(jnp.maximum(acc, jnp.float32(1e-24)))
                for d in range(D):
                    col = jnp.full((L,), d, jnp.int32)
                    plsc.store_scatter(buf, [rows, col], vs[d] * y)

        def out_slice(c):
            return out_hbm.at[pl.ds((wid * n_chunks + c) * CHUNK, CHUNK)]

        def round_body(r, carry):
            for b in range(NB):
                pltpu.async_copy(tab_hbm.at[idx_v.at[r * NB + b]], bufs[b], sin[b])
            for b in range(NB):
                c = r * NB + b
                pltpu.make_async_copy(
                    tab_hbm.at[idx_v.at[c]], bufs[b], sin[b]
                ).wait()
                normalize(bufs[b])
                pltpu.async_copy(bufs[b], out_slice(c), sout[b])
            for b in range(NB):
                pltpu.make_async_copy(bufs[b], out_slice(r * NB + b), sout[b]).wait()
            return carry

        lax.fori_loop(0, n_rounds, round_body, jnp.int32(0))

    out = run(xf, weight)
    return out.reshape(B0, S, D)
